# baseline (speedup 1.0000x reference)
import functools

import jax
import jax.numpy as jnp
from jax import lax
from jax.experimental import pallas as pl
from jax.experimental.pallas import tpu as pltpu
from jax.experimental.pallas import tpu_sc as plsc

N = 10000
F = 128
E = 320000
NCFG = 33
NPAD = 10240
NW = 32
NSUB = 16
EPW = E // NW
CHK = 128
NCHK = 80
CPCH = 16
NB = 2
EPADW = NCHK * CHK
ROWS_PT = NPAD // NSUB
BLK = 1280
GRID = NPAD // BLK

_PREC = None


def _lrelu(v):
    return jnp.where(v >= 0, v, 0.01 * v)


def _sc_mesh():
    return plsc.VectorSubcoreMesh(core_axis_name="c", subcore_axis_name="s")


def _sc_deg_body(dst_hbm, out_hbm, idx_v, ones_v, z_v, sh_deg):
    cid = lax.axis_index("c")
    sid = lax.axis_index("s")
    g = cid * NSUB + sid
    for i in range(ROWS_PT // 16):
        z_v[pl.ds(i * 16, 16)] = jnp.zeros((16,), jnp.float32)
    for i in range(CHK // 16):
        ones_v[pl.ds(i * 16, 16)] = jnp.ones((16,), jnp.float32)
    pltpu.sync_copy(z_v, sh_deg.at[pl.ds(sid * ROWS_PT, ROWS_PT)])
    pltpu.sync_copy(dst_hbm.at[g], idx_v)
    plsc.subcore_barrier()

    def body(j, carry):
        pltpu.sync_copy(ones_v, sh_deg.at[idx_v.at[j]], add=True)
        return carry

    lax.fori_loop(0, NCHK, body, 0)
    plsc.subcore_barrier()
    pltpu.sync_copy(
        sh_deg.at[pl.ds(sid * ROWS_PT, ROWS_PT)],
        out_hbm.at[cid, pl.ds(sid * ROWS_PT, ROWS_PT)],
    )


def _sc_deg(dst_p):
    k = pl.kernel(
        _sc_deg_body,
        out_type=jax.ShapeDtypeStruct((2, NPAD), jnp.float32),
        mesh=_sc_mesh(),
        scratch_types=[
            pltpu.VMEM((NCHK, CHK), jnp.int32),
            pltpu.VMEM((CHK,), jnp.float32),
            pltpu.VMEM((ROWS_PT,), jnp.float32),
            pltpu.VMEM_SHARED((NPAD,), jnp.float32),
        ],
    )
    return k(dst_p)


def _sc_spmm_body(mt_hbm, src_hbm, dst_hbm, out_hbm, sidx, didx,
                  gb0, gb1, sh_acc,
                  gs0, gs1, ss0, ss1):
    cid = lax.axis_index("c")
    sid = lax.axis_index("s")
    g = cid * NSUB + sid
    bufs = (gb0, gb1)
    gsems = (gs0, gs1)
    ssems = (ss0, ss1)
    for r in range(16):
        for c in range(F // 16):
            gb0[r, pl.ds(c * 16, 16)] = jnp.zeros((16,), jnp.float32)

    def zbody(k, carry):
        pltpu.sync_copy(gb0.at[pl.ds(0, 16)],
                        sh_acc.at[pl.ds(sid * ROWS_PT + k * 16, 16)])
        return carry

    lax.fori_loop(0, ROWS_PT // 16, zbody, 0)
    plsc.subcore_barrier()

    groups = CPCH // NB
    for phase in range(NCHK // CPCH):
        pltpu.sync_copy(src_hbm.at[g, pl.ds(phase * CPCH, CPCH)], sidx)
        pltpu.sync_copy(dst_hbm.at[g, pl.ds(phase * CPCH, CPCH)], didx)
        for b in range(NB):
            pltpu.async_copy(mt_hbm.at[sidx.at[b]], bufs[b], gsems[b])

        def body(k, carry):
            for b in range(NB):
                c = k * NB + b
                pltpu.make_async_copy(mt_hbm.at[sidx.at[c]], bufs[b],
                                      gsems[b]).wait()
                pltpu.async_copy(bufs[b], sh_acc.at[didx.at[c]], ssems[b],
                                 add=True)
            for b in range(NB):
                c = k * NB + b

                @pl.when(k < groups - 1)
                def _():
                    pltpu.make_async_copy(bufs[b], sh_acc.at[didx.at[c]],
                                          ssems[b]).wait()
                    pltpu.async_copy(mt_hbm.at[sidx.at[c + NB]], bufs[b],
                                     gsems[b])

            return carry

        lax.fori_loop(0, groups, body, 0)
        for b in range(NB):
            c = (groups - 1) * NB + b
            pltpu.make_async_copy(bufs[b], sh_acc.at[didx.at[c]],
                                  ssems[b]).wait()
    plsc.subcore_barrier()
    pltpu.sync_copy(
        sh_acc.at[pl.ds(sid * ROWS_PT, ROWS_PT)],
        out_hbm.at[cid, pl.ds(sid * ROWS_PT, ROWS_PT)],
    )


def _sc_spmm(mt, src_p, dst_p):
    k = pl.kernel(
        _sc_spmm_body,
        out_type=jax.ShapeDtypeStruct((2, NPAD, F), jnp.float32),
        mesh=_sc_mesh(),
        scratch_types=[
            pltpu.VMEM((CPCH, CHK), jnp.int32),
            pltpu.VMEM((CPCH, CHK), jnp.int32),
            pltpu.VMEM((CHK, F), jnp.float32),
            pltpu.VMEM((CHK, F), jnp.float32),
            pltpu.VMEM_SHARED((NPAD, F), jnp.float32),
            pltpu.SemaphoreType.DMA,
            pltpu.SemaphoreType.DMA,
            pltpu.SemaphoreType.DMA,
            pltpu.SemaphoreType.DMA,
        ],
    )
    return k(mt, src_p, dst_p)


def _row_mask(i, m):
    rows = i * BLK + lax.broadcasted_iota(jnp.int32, (BLK, 1), 0)
    return jnp.where(rows < N, m, 0.0)


def _tc_prenet_body(x_ref, w0_ref, b0_ref, w1_ref, b1_ref, wc_ref, dinv_ref,
                    h_ref, mt_ref):
    x = x_ref[...]
    t = _lrelu(jnp.dot(x, w0_ref[...], precision=_PREC,
                       preferred_element_type=jnp.float32) + b0_ref[...])
    h = _lrelu(jnp.dot(t, w1_ref[...], precision=_PREC,
                       preferred_element_type=jnp.float32) + b1_ref[...])
    m = jnp.dot(h, wc_ref[...], precision=_PREC,
                preferred_element_type=jnp.float32)
    dinv = dinv_ref[...]
    h_ref[...] = h
    mt_ref[...] = _row_mask(pl.program_id(0), m * dinv)


def _tc_prenet(x_p, W0, b0, W1, b1, Wc1, dinv):
    full = lambda shape: pl.BlockSpec(shape, lambda i: (0,) * len(shape))
    return pl.pallas_call(
        _tc_prenet_body,
        grid=(GRID,),
        in_specs=[
            pl.BlockSpec((BLK, F), lambda i: (i, 0)),
            full((F, 256)), full((256,)), full((256, F)), full((F,)),
            full((F, F)),
            pl.BlockSpec((BLK, 1), lambda i: (i, 0)),
        ],
        out_specs=[
            pl.BlockSpec((BLK, F), lambda i: (i, 0)),
            pl.BlockSpec((BLK, F), lambda i: (i, 0)),
        ],
        out_shape=[
            jax.ShapeDtypeStruct((NPAD, F), jnp.float32),
            jax.ShapeDtypeStruct((NPAD, F), jnp.float32),
        ],
    )(x_p, W0, b0, W1, b1, Wc1, dinv)


def _tc_mid_body(parts_ref, mt_ref, h_ref, bc_ref, wc_ref, dinv_ref,
                 hout_ref, mtout_ref):
    s = parts_ref[0] + parts_ref[1] + mt_ref[...]
    dinv = dinv_ref[...]
    h = _lrelu(s * dinv + bc_ref[...]) + h_ref[...]
    m = jnp.dot(h, wc_ref[...], precision=_PREC,
                preferred_element_type=jnp.float32)
    hout_ref[...] = h
    mtout_ref[...] = _row_mask(pl.program_id(0), m * dinv)


def _tc_mid(parts, mt, h, bc, wc_next, dinv):
    full = lambda shape: pl.BlockSpec(shape, lambda i: (0,) * len(shape))
    return pl.pallas_call(
        _tc_mid_body,
        grid=(GRID,),
        in_specs=[
            pl.BlockSpec((2, BLK, F), lambda i: (0, i, 0)),
            pl.BlockSpec((BLK, F), lambda i: (i, 0)),
            pl.BlockSpec((BLK, F), lambda i: (i, 0)),
            full((F,)), full((F, F)),
            pl.BlockSpec((BLK, 1), lambda i: (i, 0)),
        ],
        out_specs=[
            pl.BlockSpec((BLK, F), lambda i: (i, 0)),
            pl.BlockSpec((BLK, F), lambda i: (i, 0)),
        ],
        out_shape=[
            jax.ShapeDtypeStruct((NPAD, F), jnp.float32),
            jax.ShapeDtypeStruct((NPAD, F), jnp.float32),
        ],
    )(parts, mt, h, bc, wc_next, dinv)


def _tc_final_body(parts_ref, mt_ref, h_ref, bc_ref, dinv_ref, bid_ref,
                   wpost_ref, out_ref, acc_ref):
    i = pl.program_id(0)
    s = parts_ref[0] + parts_ref[1] + mt_ref[...]
    h = _lrelu(s * dinv_ref[...] + bc_ref[...]) + h_ref[...]
    bid = bid_ref[...]
    neg = jnp.float32(-jnp.inf)
    pooled = jnp.stack(
        [jnp.max(jnp.where(bid == c, h, neg), axis=0) for c in range(NCFG)]
    )

    @pl.when(i == 0)
    def _():
        acc_ref[...] = pooled

    @pl.when(i > 0)
    def _():
        acc_ref[...] = jnp.maximum(acc_ref[...], pooled)

    @pl.when(i == GRID - 1)
    def _():
        w = wpost_ref[...]
        pred = jnp.dot(acc_ref[...], w,
                       preferred_element_type=jnp.float32)
        out_ref[...] = pred.reshape(1, NCFG)


def _tc_final(parts, mt, h, bc, dinv, bid_p, Wpost):
    full = lambda shape: pl.BlockSpec(shape, lambda i: (0,) * len(shape))
    return pl.pallas_call(
        _tc_final_body,
        grid=(GRID,),
        in_specs=[
            pl.BlockSpec((2, BLK, F), lambda i: (0, i, 0)),
            pl.BlockSpec((BLK, F), lambda i: (i, 0)),
            pl.BlockSpec((BLK, F), lambda i: (i, 0)),
            full((F,)),
            pl.BlockSpec((BLK, 1), lambda i: (i, 0)),
            pl.BlockSpec((BLK, 1), lambda i: (i, 0)),
            full((F, 1)),
        ],
        out_specs=pl.BlockSpec((1, NCFG), lambda i: (0, 0)),
        out_shape=jax.ShapeDtypeStruct((1, NCFG), jnp.float32),
        scratch_shapes=[pltpu.VMEM((NCFG, F), jnp.float32)],
    )(parts, mt, h, bc, dinv, bid_p, Wpost)


def kernel(x, W0, b0, W1, b1, Wc1, bc1, Wc2, bc2, Wc3, bc3, Wpost,
           edge_index, batch_ids):
    src = edge_index[0].astype(jnp.int32)
    dst = edge_index[1].astype(jnp.int32)
    pad_cols = EPADW - EPW
    padi = jnp.full((NW, pad_cols), N, jnp.int32)
    src_p = jnp.concatenate([src.reshape(NW, EPW), padi], axis=1)
    src_p = src_p.reshape(NW, NCHK, CHK)
    dst_p = jnp.concatenate([dst.reshape(NW, EPW), padi], axis=1)
    dst_p = dst_p.reshape(NW, NCHK, CHK)

    x_p = jnp.pad(x, ((0, NPAD - N), (0, 0)))
    bid_p = jnp.pad(batch_ids.astype(jnp.int32), (0, NPAD - N),
                    constant_values=NCFG).reshape(NPAD, 1)

    deg_parts = _sc_deg(dst_p)
    dinv = lax.rsqrt(1.0 + deg_parts[0] + deg_parts[1]).reshape(NPAD, 1)

    h0, mt1 = _tc_prenet(x_p, W0, b0, W1, b1, Wc1, dinv)
    p1 = _sc_spmm(mt1, src_p, dst_p)
    h1, mt2 = _tc_mid(p1, mt1, h0, bc1, Wc2, dinv)
    p2 = _sc_spmm(mt2, src_p, dst_p)
    h2, mt3 = _tc_mid(p2, mt2, h1, bc2, Wc3, dinv)
    p3 = _sc_spmm(mt3, src_p, dst_p)
    return _tc_final(p3, mt3, h2, bc3, dinv, bid_p, Wpost)


# speedup vs baseline: 1.0461x; 1.0461x over previous
import functools

import jax
import jax.numpy as jnp
from jax import lax
from jax.experimental import pallas as pl
from jax.experimental.pallas import tpu as pltpu
from jax.experimental.pallas import tpu_sc as plsc

N = 10000
F = 128
E = 320000
NCFG = 33
NPAD = 10240
NW = 32
NSUB = 16
EPW = E // NW
CHK = 32
NCHK = 320
CPCH = 64
NB = 8
EPADW = NCHK * CHK
ROWS_PT = NPAD // NSUB
BLK = 1280
GRID = NPAD // BLK

_PREC = None


def _lrelu(v):
    return jnp.where(v >= 0, v, 0.01 * v)


def _sc_mesh():
    return plsc.VectorSubcoreMesh(core_axis_name="c", subcore_axis_name="s")


def _sc_deg_body(dst_hbm, out_hbm, idx_v, ones_v, z_v, sh_deg):
    cid = lax.axis_index("c")
    sid = lax.axis_index("s")
    g = cid * NSUB + sid
    for i in range(ROWS_PT // 16):
        z_v[pl.ds(i * 16, 16)] = jnp.zeros((16,), jnp.float32)
    for i in range(CHK // 16):
        ones_v[pl.ds(i * 16, 16)] = jnp.ones((16,), jnp.float32)
    pltpu.sync_copy(z_v, sh_deg.at[pl.ds(sid * ROWS_PT, ROWS_PT)])
    pltpu.sync_copy(dst_hbm.at[g], idx_v)
    plsc.subcore_barrier()

    def body(j, carry):
        pltpu.sync_copy(ones_v, sh_deg.at[idx_v.at[j]], add=True)
        return carry

    lax.fori_loop(0, NCHK, body, 0)
    plsc.subcore_barrier()
    pltpu.sync_copy(
        sh_deg.at[pl.ds(sid * ROWS_PT, ROWS_PT)],
        out_hbm.at[cid, pl.ds(sid * ROWS_PT, ROWS_PT)],
    )


def _sc_deg(dst_p):
    k = pl.kernel(
        _sc_deg_body,
        out_type=jax.ShapeDtypeStruct((2, NPAD), jnp.float32),
        mesh=_sc_mesh(),
        scratch_types=[
            pltpu.VMEM((NCHK, CHK), jnp.int32),
            pltpu.VMEM((CHK,), jnp.float32),
            pltpu.VMEM((ROWS_PT,), jnp.float32),
            pltpu.VMEM_SHARED((NPAD,), jnp.float32),
        ],
    )
    return k(dst_p)


def _sc_spmm_body(mt_hbm, src_hbm, dst_hbm, out_hbm, sidx, didx,
                  gb0, gb1, gb2, gb3, gb4, gb5, gb6, gb7, sh_acc,
                  gs0, gs1, gs2, gs3, gs4, gs5, gs6, gs7,
                  ss0, ss1, ss2, ss3, ss4, ss5, ss6, ss7):
    cid = lax.axis_index("c")
    sid = lax.axis_index("s")
    g = cid * NSUB + sid
    bufs = (gb0, gb1, gb2, gb3, gb4, gb5, gb6, gb7)
    gsems = (gs0, gs1, gs2, gs3, gs4, gs5, gs6, gs7)
    ssems = (ss0, ss1, ss2, ss3, ss4, ss5, ss6, ss7)
    for r in range(16):
        for c in range(F // 16):
            gb0[r, pl.ds(c * 16, 16)] = jnp.zeros((16,), jnp.float32)

    def zbody(k, carry):
        pltpu.sync_copy(gb0.at[pl.ds(0, 16)],
                        sh_acc.at[pl.ds(sid * ROWS_PT + k * 16, 16)])
        return carry

    lax.fori_loop(0, ROWS_PT // 16, zbody, 0)
    plsc.subcore_barrier()

    groups = CPCH // NB
    for phase in range(NCHK // CPCH):
        pltpu.sync_copy(src_hbm.at[g, pl.ds(phase * CPCH, CPCH)], sidx)
        pltpu.sync_copy(dst_hbm.at[g, pl.ds(phase * CPCH, CPCH)], didx)
        for b in range(NB):
            pltpu.async_copy(mt_hbm.at[sidx.at[b]], bufs[b], gsems[b])

        def body(k, carry):
            for b in range(NB):
                c = k * NB + b
                pltpu.make_async_copy(mt_hbm.at[sidx.at[c]], bufs[b],
                                      gsems[b]).wait()
                pltpu.async_copy(bufs[b], sh_acc.at[didx.at[c]], ssems[b],
                                 add=True)
            for b in range(NB):
                c = k * NB + b

                @pl.when(k < groups - 1)
                def _():
                    pltpu.make_async_copy(bufs[b], sh_acc.at[didx.at[c]],
                                          ssems[b]).wait()
                    pltpu.async_copy(mt_hbm.at[sidx.at[c + NB]], bufs[b],
                                     gsems[b])

            return carry

        lax.fori_loop(0, groups, body, 0)
        for b in range(NB):
            c = (groups - 1) * NB + b
            pltpu.make_async_copy(bufs[b], sh_acc.at[didx.at[c]],
                                  ssems[b]).wait()
    plsc.subcore_barrier()
    pltpu.sync_copy(
        sh_acc.at[pl.ds(sid * ROWS_PT, ROWS_PT)],
        out_hbm.at[cid, pl.ds(sid * ROWS_PT, ROWS_PT)],
    )


def _sc_spmm(mt, src_p, dst_p):
    k = pl.kernel(
        _sc_spmm_body,
        out_type=jax.ShapeDtypeStruct((2, NPAD, F), jnp.float32),
        mesh=_sc_mesh(),
        scratch_types=[
            pltpu.VMEM((CPCH, CHK), jnp.int32),
            pltpu.VMEM((CPCH, CHK), jnp.int32),
            pltpu.VMEM((CHK, F), jnp.float32),
            pltpu.VMEM((CHK, F), jnp.float32),
            pltpu.VMEM((CHK, F), jnp.float32),
            pltpu.VMEM((CHK, F), jnp.float32),
            pltpu.VMEM((CHK, F), jnp.float32),
            pltpu.VMEM((CHK, F), jnp.float32),
            pltpu.VMEM((CHK, F), jnp.float32),
            pltpu.VMEM((CHK, F), jnp.float32),
            pltpu.VMEM_SHARED((NPAD, F), jnp.float32),
            pltpu.SemaphoreType.DMA,
            pltpu.SemaphoreType.DMA,
            pltpu.SemaphoreType.DMA,
            pltpu.SemaphoreType.DMA,
            pltpu.SemaphoreType.DMA,
            pltpu.SemaphoreType.DMA,
            pltpu.SemaphoreType.DMA,
            pltpu.SemaphoreType.DMA,
            pltpu.SemaphoreType.DMA,
            pltpu.SemaphoreType.DMA,
            pltpu.SemaphoreType.DMA,
            pltpu.SemaphoreType.DMA,
            pltpu.SemaphoreType.DMA,
            pltpu.SemaphoreType.DMA,
            pltpu.SemaphoreType.DMA,
            pltpu.SemaphoreType.DMA,
        ],
    )
    return k(mt, src_p, dst_p)


def _row_mask(i, m):
    rows = i * BLK + lax.broadcasted_iota(jnp.int32, (BLK, 1), 0)
    return jnp.where(rows < N, m, 0.0)


def _tc_prenet_body(x_ref, w0_ref, b0_ref, w1_ref, b1_ref, wc_ref, dinv_ref,
                    h_ref, mt_ref):
    x = x_ref[...]
    t = _lrelu(jnp.dot(x, w0_ref[...], precision=_PREC,
                       preferred_element_type=jnp.float32) + b0_ref[...])
    h = _lrelu(jnp.dot(t, w1_ref[...], precision=_PREC,
                       preferred_element_type=jnp.float32) + b1_ref[...])
    m = jnp.dot(h, wc_ref[...], precision=_PREC,
                preferred_element_type=jnp.float32)
    dinv = dinv_ref[...]
    h_ref[...] = h
    mt_ref[...] = _row_mask(pl.program_id(0), m * dinv)


def _tc_prenet(x_p, W0, b0, W1, b1, Wc1, dinv):
    full = lambda shape: pl.BlockSpec(shape, lambda i: (0,) * len(shape))
    return pl.pallas_call(
        _tc_prenet_body,
        grid=(GRID,),
        in_specs=[
            pl.BlockSpec((BLK, F), lambda i: (i, 0)),
            full((F, 256)), full((256,)), full((256, F)), full((F,)),
            full((F, F)),
            pl.BlockSpec((BLK, 1), lambda i: (i, 0)),
        ],
        out_specs=[
            pl.BlockSpec((BLK, F), lambda i: (i, 0)),
            pl.BlockSpec((BLK, F), lambda i: (i, 0)),
        ],
        out_shape=[
            jax.ShapeDtypeStruct((NPAD, F), jnp.float32),
            jax.ShapeDtypeStruct((NPAD, F), jnp.float32),
        ],
    )(x_p, W0, b0, W1, b1, Wc1, dinv)


def _tc_mid_body(parts_ref, mt_ref, h_ref, bc_ref, wc_ref, dinv_ref,
                 hout_ref, mtout_ref):
    s = parts_ref[0] + parts_ref[1] + mt_ref[...]
    dinv = dinv_ref[...]
    h = _lrelu(s * dinv + bc_ref[...]) + h_ref[...]
    m = jnp.dot(h, wc_ref[...], precision=_PREC,
                preferred_element_type=jnp.float32)
    hout_ref[...] = h
    mtout_ref[...] = _row_mask(pl.program_id(0), m * dinv)


def _tc_mid(parts, mt, h, bc, wc_next, dinv):
    full = lambda shape: pl.BlockSpec(shape, lambda i: (0,) * len(shape))
    return pl.pallas_call(
        _tc_mid_body,
        grid=(GRID,),
        in_specs=[
            pl.BlockSpec((2, BLK, F), lambda i: (0, i, 0)),
            pl.BlockSpec((BLK, F), lambda i: (i, 0)),
            pl.BlockSpec((BLK, F), lambda i: (i, 0)),
            full((F,)), full((F, F)),
            pl.BlockSpec((BLK, 1), lambda i: (i, 0)),
        ],
        out_specs=[
            pl.BlockSpec((BLK, F), lambda i: (i, 0)),
            pl.BlockSpec((BLK, F), lambda i: (i, 0)),
        ],
        out_shape=[
            jax.ShapeDtypeStruct((NPAD, F), jnp.float32),
            jax.ShapeDtypeStruct((NPAD, F), jnp.float32),
        ],
    )(parts, mt, h, bc, wc_next, dinv)


def _tc_final_body(parts_ref, mt_ref, h_ref, bc_ref, dinv_ref, bid_ref,
                   wpost_ref, out_ref, acc_ref):
    i = pl.program_id(0)
    s = parts_ref[0] + parts_ref[1] + mt_ref[...]
    h = _lrelu(s * dinv_ref[...] + bc_ref[...]) + h_ref[...]
    bid = bid_ref[...]
    neg = jnp.float32(-jnp.inf)
    pooled = jnp.stack(
        [jnp.max(jnp.where(bid == c, h, neg), axis=0) for c in range(NCFG)]
    )

    @pl.when(i == 0)
    def _():
        acc_ref[...] = pooled

    @pl.when(i > 0)
    def _():
        acc_ref[...] = jnp.maximum(acc_ref[...], pooled)

    @pl.when(i == GRID - 1)
    def _():
        w = wpost_ref[...]
        pred = jnp.dot(acc_ref[...], w,
                       preferred_element_type=jnp.float32)
        out_ref[...] = pred.reshape(1, NCFG)


def _tc_final(parts, mt, h, bc, dinv, bid_p, Wpost):
    full = lambda shape: pl.BlockSpec(shape, lambda i: (0,) * len(shape))
    return pl.pallas_call(
        _tc_final_body,
        grid=(GRID,),
        in_specs=[
            pl.BlockSpec((2, BLK, F), lambda i: (0, i, 0)),
            pl.BlockSpec((BLK, F), lambda i: (i, 0)),
            pl.BlockSpec((BLK, F), lambda i: (i, 0)),
            full((F,)),
            pl.BlockSpec((BLK, 1), lambda i: (i, 0)),
            pl.BlockSpec((BLK, 1), lambda i: (i, 0)),
            full((F, 1)),
        ],
        out_specs=pl.BlockSpec((1, NCFG), lambda i: (0, 0)),
        out_shape=jax.ShapeDtypeStruct((1, NCFG), jnp.float32),
        scratch_shapes=[pltpu.VMEM((NCFG, F), jnp.float32)],
    )(parts, mt, h, bc, dinv, bid_p, Wpost)


def kernel(x, W0, b0, W1, b1, Wc1, bc1, Wc2, bc2, Wc3, bc3, Wpost,
           edge_index, batch_ids):
    src = edge_index[0].astype(jnp.int32)
    dst = edge_index[1].astype(jnp.int32)
    pad_cols = EPADW - EPW
    padi = jnp.full((NW, pad_cols), N, jnp.int32)
    src_p = jnp.concatenate([src.reshape(NW, EPW), padi], axis=1)
    src_p = src_p.reshape(NW, NCHK, CHK)
    dst_p = jnp.concatenate([dst.reshape(NW, EPW), padi], axis=1)
    dst_p = dst_p.reshape(NW, NCHK, CHK)

    x_p = jnp.pad(x, ((0, NPAD - N), (0, 0)))
    bid_p = jnp.pad(batch_ids.astype(jnp.int32), (0, NPAD - N),
                    constant_values=NCFG).reshape(NPAD, 1)

    deg_parts = _sc_deg(dst_p)
    dinv = lax.rsqrt(1.0 + deg_parts[0] + deg_parts[1]).reshape(NPAD, 1)

    h0, mt1 = _tc_prenet(x_p, W0, b0, W1, b1, Wc1, dinv)
    p1 = _sc_spmm(mt1, src_p, dst_p)
    h1, mt2 = _tc_mid(p1, mt1, h0, bc1, Wc2, dinv)
    p2 = _sc_spmm(mt2, src_p, dst_p)
    h2, mt3 = _tc_mid(p2, mt2, h1, bc2, Wc3, dinv)
    p3 = _sc_spmm(mt3, src_p, dst_p)
    return _tc_final(p3, mt3, h2, bc3, dinv, bid_p, Wpost)


# speedup vs baseline: 1.0531x; 1.0067x over previous
import functools

import jax
import jax.numpy as jnp
from jax import lax
from jax.experimental import pallas as pl
from jax.experimental.pallas import tpu as pltpu
from jax.experimental.pallas import tpu_sc as plsc

N = 10000
F = 128
E = 320000
NCFG = 33
NPAD = 10240
NW = 32
NSUB = 16
EPW = E // NW
CHK = 64
NCHK = 160
CPCH = 32
NB = 4
EPADW = NCHK * CHK
ROWS_PT = NPAD // NSUB
BLK = 1280
GRID = NPAD // BLK

_PREC = None


def _lrelu(v):
    return jnp.where(v >= 0, v, 0.01 * v)


def _sc_mesh():
    return plsc.VectorSubcoreMesh(core_axis_name="c", subcore_axis_name="s")


def _sc_deg_body(dst_hbm, out_hbm, idx_v, ones_v, z_v, sh_deg):
    cid = lax.axis_index("c")
    sid = lax.axis_index("s")
    g = cid * NSUB + sid
    for i in range(ROWS_PT // 16):
        z_v[pl.ds(i * 16, 16)] = jnp.zeros((16,), jnp.float32)
    for i in range(CHK // 16):
        ones_v[pl.ds(i * 16, 16)] = jnp.ones((16,), jnp.float32)
    pltpu.sync_copy(z_v, sh_deg.at[pl.ds(sid * ROWS_PT, ROWS_PT)])
    pltpu.sync_copy(dst_hbm.at[g], idx_v)
    plsc.subcore_barrier()

    def body(j, carry):
        pltpu.sync_copy(ones_v, sh_deg.at[idx_v.at[j]], add=True)
        return carry

    lax.fori_loop(0, NCHK, body, 0)
    plsc.subcore_barrier()
    pltpu.sync_copy(
        sh_deg.at[pl.ds(sid * ROWS_PT, ROWS_PT)],
        out_hbm.at[cid, pl.ds(sid * ROWS_PT, ROWS_PT)],
    )


def _sc_deg(dst_p):
    k = pl.kernel(
        _sc_deg_body,
        out_type=jax.ShapeDtypeStruct((2, NPAD), jnp.float32),
        mesh=_sc_mesh(),
        scratch_types=[
            pltpu.VMEM((NCHK, CHK), jnp.int32),
            pltpu.VMEM((CHK,), jnp.float32),
            pltpu.VMEM((ROWS_PT,), jnp.float32),
            pltpu.VMEM_SHARED((NPAD,), jnp.float32),
        ],
    )
    return k(dst_p)


def _sc_spmm_body(mt_hbm, src_hbm, dst_hbm, out_hbm, sidx, didx,
                  gb0, gb1, gb2, gb3, sh_acc,
                  gs0, gs1, gs2, gs3, ss0, ss1, ss2, ss3):
    cid = lax.axis_index("c")
    sid = lax.axis_index("s")
    g = cid * NSUB + sid
    bufs = (gb0, gb1, gb2, gb3)
    gsems = (gs0, gs1, gs2, gs3)
    ssems = (ss0, ss1, ss2, ss3)
    for r in range(16):
        for c in range(F // 16):
            gb0[r, pl.ds(c * 16, 16)] = jnp.zeros((16,), jnp.float32)

    def zbody(k, carry):
        pltpu.sync_copy(gb0.at[pl.ds(0, 16)],
                        sh_acc.at[pl.ds(sid * ROWS_PT + k * 16, 16)])
        return carry

    lax.fori_loop(0, ROWS_PT // 16, zbody, 0)
    plsc.subcore_barrier()

    groups = CPCH // NB
    for phase in range(NCHK // CPCH):
        pltpu.sync_copy(src_hbm.at[g, pl.ds(phase * CPCH, CPCH)], sidx)
        pltpu.sync_copy(dst_hbm.at[g, pl.ds(phase * CPCH, CPCH)], didx)
        for b in range(NB):
            pltpu.async_copy(mt_hbm.at[sidx.at[b]], bufs[b], gsems[b])

        def body(k, carry):
            for b in range(NB):
                c = k * NB + b
                pltpu.make_async_copy(mt_hbm.at[sidx.at[c]], bufs[b],
                                      gsems[b]).wait()
                pltpu.async_copy(bufs[b], sh_acc.at[didx.at[c]], ssems[b],
                                 add=True)
            for b in range(NB):
                c = k * NB + b

                @pl.when(k < groups - 1)
                def _():
                    pltpu.make_async_copy(bufs[b], sh_acc.at[didx.at[c]],
                                          ssems[b]).wait()
                    pltpu.async_copy(mt_hbm.at[sidx.at[c + NB]], bufs[b],
                                     gsems[b])

            return carry

        lax.fori_loop(0, groups, body, 0)
        for b in range(NB):
            c = (groups - 1) * NB + b
            pltpu.make_async_copy(bufs[b], sh_acc.at[didx.at[c]],
                                  ssems[b]).wait()
    plsc.subcore_barrier()
    pltpu.sync_copy(
        sh_acc.at[pl.ds(sid * ROWS_PT, ROWS_PT)],
        out_hbm.at[cid, pl.ds(sid * ROWS_PT, ROWS_PT)],
    )


def _sc_spmm(mt, src_p, dst_p):
    k = pl.kernel(
        _sc_spmm_body,
        out_type=jax.ShapeDtypeStruct((2, NPAD, F), jnp.float32),
        mesh=_sc_mesh(),
        scratch_types=[
            pltpu.VMEM((CPCH, CHK), jnp.int32),
            pltpu.VMEM((CPCH, CHK), jnp.int32),
            pltpu.VMEM((CHK, F), jnp.float32),
            pltpu.VMEM((CHK, F), jnp.float32),
            pltpu.VMEM((CHK, F), jnp.float32),
            pltpu.VMEM((CHK, F), jnp.float32),
            pltpu.VMEM_SHARED((NPAD, F), jnp.float32),
            pltpu.SemaphoreType.DMA,
            pltpu.SemaphoreType.DMA,
            pltpu.SemaphoreType.DMA,
            pltpu.SemaphoreType.DMA,
            pltpu.SemaphoreType.DMA,
            pltpu.SemaphoreType.DMA,
            pltpu.SemaphoreType.DMA,
            pltpu.SemaphoreType.DMA,
        ],
    )
    return k(mt, src_p, dst_p)


def _row_mask(i, m):
    rows = i * BLK + lax.broadcasted_iota(jnp.int32, (BLK, 1), 0)
    return jnp.where(rows < N, m, 0.0)


def _tc_prenet_body(x_ref, w0_ref, b0_ref, w1_ref, b1_ref, wc_ref, dinv_ref,
                    h_ref, mt_ref):
    x = x_ref[...]
    t = _lrelu(jnp.dot(x, w0_ref[...], precision=_PREC,
                       preferred_element_type=jnp.float32) + b0_ref[...])
    h = _lrelu(jnp.dot(t, w1_ref[...], precision=_PREC,
                       preferred_element_type=jnp.float32) + b1_ref[...])
    m = jnp.dot(h, wc_ref[...], precision=_PREC,
                preferred_element_type=jnp.float32)
    dinv = dinv_ref[...]
    h_ref[...] = h
    mt_ref[...] = _row_mask(pl.program_id(0), m * dinv)


def _tc_prenet(x_p, W0, b0, W1, b1, Wc1, dinv):
    full = lambda shape: pl.BlockSpec(shape, lambda i: (0,) * len(shape))
    return pl.pallas_call(
        _tc_prenet_body,
        grid=(GRID,),
        in_specs=[
            pl.BlockSpec((BLK, F), lambda i: (i, 0)),
            full((F, 256)), full((256,)), full((256, F)), full((F,)),
            full((F, F)),
            pl.BlockSpec((BLK, 1), lambda i: (i, 0)),
        ],
        out_specs=[
            pl.BlockSpec((BLK, F), lambda i: (i, 0)),
            pl.BlockSpec((BLK, F), lambda i: (i, 0)),
        ],
        out_shape=[
            jax.ShapeDtypeStruct((NPAD, F), jnp.float32),
            jax.ShapeDtypeStruct((NPAD, F), jnp.float32),
        ],
    )(x_p, W0, b0, W1, b1, Wc1, dinv)


def _tc_mid_body(parts_ref, mt_ref, h_ref, bc_ref, wc_ref, dinv_ref,
                 hout_ref, mtout_ref):
    s = parts_ref[0] + parts_ref[1] + mt_ref[...]
    dinv = dinv_ref[...]
    h = _lrelu(s * dinv + bc_ref[...]) + h_ref[...]
    m = jnp.dot(h, wc_ref[...], precision=_PREC,
                preferred_element_type=jnp.float32)
    hout_ref[...] = h
    mtout_ref[...] = _row_mask(pl.program_id(0), m * dinv)


def _tc_mid(parts, mt, h, bc, wc_next, dinv):
    full = lambda shape: pl.BlockSpec(shape, lambda i: (0,) * len(shape))
    return pl.pallas_call(
        _tc_mid_body,
        grid=(GRID,),
        in_specs=[
            pl.BlockSpec((2, BLK, F), lambda i: (0, i, 0)),
            pl.BlockSpec((BLK, F), lambda i: (i, 0)),
            pl.BlockSpec((BLK, F), lambda i: (i, 0)),
            full((F,)), full((F, F)),
            pl.BlockSpec((BLK, 1), lambda i: (i, 0)),
        ],
        out_specs=[
            pl.BlockSpec((BLK, F), lambda i: (i, 0)),
            pl.BlockSpec((BLK, F), lambda i: (i, 0)),
        ],
        out_shape=[
            jax.ShapeDtypeStruct((NPAD, F), jnp.float32),
            jax.ShapeDtypeStruct((NPAD, F), jnp.float32),
        ],
    )(parts, mt, h, bc, wc_next, dinv)


def _tc_final_body(parts_ref, mt_ref, h_ref, bc_ref, dinv_ref, bid_ref,
                   wpost_ref, out_ref, acc_ref):
    i = pl.program_id(0)
    s = parts_ref[0] + parts_ref[1] + mt_ref[...]
    h = _lrelu(s * dinv_ref[...] + bc_ref[...]) + h_ref[...]
    bid = bid_ref[...]
    neg = jnp.float32(-jnp.inf)
    pooled = jnp.stack(
        [jnp.max(jnp.where(bid == c, h, neg), axis=0) for c in range(NCFG)]
    )

    @pl.when(i == 0)
    def _():
        acc_ref[...] = pooled

    @pl.when(i > 0)
    def _():
        acc_ref[...] = jnp.maximum(acc_ref[...], pooled)

    @pl.when(i == GRID - 1)
    def _():
        w = wpost_ref[...]
        pred = jnp.dot(acc_ref[...], w,
                       preferred_element_type=jnp.float32)
        out_ref[...] = pred.reshape(1, NCFG)


def _tc_final(parts, mt, h, bc, dinv, bid_p, Wpost):
    full = lambda shape: pl.BlockSpec(shape, lambda i: (0,) * len(shape))
    return pl.pallas_call(
        _tc_final_body,
        grid=(GRID,),
        in_specs=[
            pl.BlockSpec((2, BLK, F), lambda i: (0, i, 0)),
            pl.BlockSpec((BLK, F), lambda i: (i, 0)),
            pl.BlockSpec((BLK, F), lambda i: (i, 0)),
            full((F,)),
            pl.BlockSpec((BLK, 1), lambda i: (i, 0)),
            pl.BlockSpec((BLK, 1), lambda i: (i, 0)),
            full((F, 1)),
        ],
        out_specs=pl.BlockSpec((1, NCFG), lambda i: (0, 0)),
        out_shape=jax.ShapeDtypeStruct((1, NCFG), jnp.float32),
        scratch_shapes=[pltpu.VMEM((NCFG, F), jnp.float32)],
    )(parts, mt, h, bc, dinv, bid_p, Wpost)


def kernel(x, W0, b0, W1, b1, Wc1, bc1, Wc2, bc2, Wc3, bc3, Wpost,
           edge_index, batch_ids):
    src = edge_index[0].astype(jnp.int32)
    dst = edge_index[1].astype(jnp.int32)
    pad_cols = EPADW - EPW
    padi = jnp.full((NW, pad_cols), N, jnp.int32)
    src_p = jnp.concatenate([src.reshape(NW, EPW), padi], axis=1)
    src_p = src_p.reshape(NW, NCHK, CHK)
    dst_p = jnp.concatenate([dst.reshape(NW, EPW), padi], axis=1)
    dst_p = dst_p.reshape(NW, NCHK, CHK)

    x_p = jnp.pad(x, ((0, NPAD - N), (0, 0)))
    bid_p = jnp.pad(batch_ids.astype(jnp.int32), (0, NPAD - N),
                    constant_values=NCFG).reshape(NPAD, 1)

    deg_parts = _sc_deg(dst_p)
    dinv = lax.rsqrt(1.0 + deg_parts[0] + deg_parts[1]).reshape(NPAD, 1)

    h0, mt1 = _tc_prenet(x_p, W0, b0, W1, b1, Wc1, dinv)
    p1 = _sc_spmm(mt1, src_p, dst_p)
    h1, mt2 = _tc_mid(p1, mt1, h0, bc1, Wc2, dinv)
    p2 = _sc_spmm(mt2, src_p, dst_p)
    h2, mt3 = _tc_mid(p2, mt2, h1, bc2, Wc3, dinv)
    p3 = _sc_spmm(mt3, src_p, dst_p)
    return _tc_final(p3, mt3, h2, bc3, dinv, bid_p, Wpost)
